# baseline (speedup 1.0000x reference)
import functools

import jax
import jax.numpy as jnp
from jax import lax
from jax.experimental import pallas as pl
from jax.experimental.pallas import tpu as pltpu
from jax.experimental.pallas import tpu_sc as plsc

NC = 2
NS = 16
NW = NC * NS
CHUNK = 32
RING = 4


@functools.lru_cache(maxsize=None)
def _make_lookup(Bt, S, D):
    B = Bt * S
    b_per_w = B // NW
    w_per_row = S // b_per_w
    n_chunks = b_per_w // CHUNK
    assert n_chunks > RING
    mesh = plsc.VectorSubcoreMesh(core_axis_name="c", subcore_axis_name="s")

    @functools.partial(
        pl.kernel,
        mesh=mesh,
        out_type=jax.ShapeDtypeStruct((B, D), jnp.float32),
        scratch_types=[
            pltpu.VMEM((b_per_w,), jnp.int32),
            pltpu.VMEM((RING * CHUNK, D), jnp.float32),
            pltpu.SemaphoreType.DMA,
            pltpu.SemaphoreType.DMA,
        ],
    )
    def lookup(idx_hbm, table_hbm, out_hbm, idx_v, rows, gsem, ssem):
        wid = lax.axis_index("s") * NC + lax.axis_index("c")
        base = wid * b_per_w
        pltpu.sync_copy(
            idx_hbm.at[wid // w_per_row,
                       pl.ds((wid % w_per_row) * b_per_w, b_per_w)],
            idx_v)

        def gather(j):
            slot = lax.rem(j, RING)
            pltpu.async_copy(
                table_hbm.at[idx_v.at[pl.ds(j * CHUNK, CHUNK)]],
                rows.at[pl.ds(slot * CHUNK, CHUNK)], gsem)

        def store(j):
            slot = lax.rem(j, RING)
            pltpu.async_copy(
                rows.at[pl.ds(slot * CHUNK, CHUNK)],
                out_hbm.at[pl.ds(base + j * CHUNK, CHUNK)], ssem)

        def wait_gather():
            pltpu.make_async_copy(
                table_hbm.at[pl.ds(0, CHUNK)],
                rows.at[pl.ds(0, CHUNK)], gsem).wait()

        def wait_store():
            pltpu.make_async_copy(
                rows.at[pl.ds(0, CHUNK)],
                out_hbm.at[pl.ds(base, CHUNK)], ssem).wait()

        for j in range(RING - 1):
            gather(j)

        def body(j, carry):
            wait_gather()
            store(j)

            @pl.when(j > 0)
            def _():
                wait_store()

            @pl.when(j + RING - 1 < n_chunks)
            def _():
                gather(j + RING - 1)

            return carry

        lax.fori_loop(0, n_chunks, body, 0)
        wait_store()

    return lookup


def kernel(input_ids, embed):
    Bt, S = input_ids.shape
    D = embed.shape[1]
    ids = input_ids.astype(jnp.int32)
    out = _make_lookup(Bt, S, D)(ids, embed)
    return out.reshape(Bt, S, D)
